# Initial kernel scaffold; baseline (speedup 1.0000x reference)
import jax
import jax.numpy as jnp
from jax import lax
from jax.experimental import pallas as pl
from jax.experimental.pallas import tpu as pltpu
from jax.experimental.pallas import tpu_sc as plsc

N = 10000
E = 320000
D = 128
ED = 16

NC = 2
NS = 16
NW = NC * NS
EPW = E // NW
B = 80
NB = EPW // B
RPT = N // NS


def _edge_proj_body(ea_ref, we_ref, be_ref, out_ref):
    out_ref[...] = (
        jnp.dot(ea_ref[...], we_ref[...], preferred_element_type=jnp.float32)
        + be_ref[...]
    )


def _edge_proj(edge_attr, We, be):
    BE = 4000
    return pl.pallas_call(
        _edge_proj_body,
        grid=(E // BE,),
        in_specs=[
            pl.BlockSpec((BE, ED), lambda i: (i, 0)),
            pl.BlockSpec((ED, D), lambda i: (0, 0)),
            pl.BlockSpec((1, D), lambda i: (0, 0)),
        ],
        out_specs=pl.BlockSpec((BE, D), lambda i: (i, 0)),
        out_shape=jax.ShapeDtypeStruct((E, D), jnp.float32),
    )(edge_attr, We, be.reshape(1, D))


def _sc_agg_body(x_hbm, e_hbm, src_hbm, dst_hbm, zero_hbm, out_hbm,
                 srcv, dstv, ev, xv, accum, sem):
    c = lax.axis_index("c")
    s = lax.axis_index("s")
    wid = c * NS + s

    pltpu.sync_copy(zero_hbm, accum.at[pl.ds(s * RPT, RPT)])
    plsc.subcore_barrier()

    ebase = pl.multiple_of(wid * EPW, 16)

    def block(j, carry):
        base = pl.multiple_of(ebase + j * B, 16)
        pltpu.sync_copy(src_hbm.at[pl.ds(base, B)], srcv)
        pltpu.sync_copy(dst_hbm.at[pl.ds(base, B)], dstv)
        pltpu.sync_copy(e_hbm.at[pl.ds(base, B)], ev)
        pltpu.async_copy(x_hbm.at[srcv], xv, sem).wait()

        def edge(i, c2):
            for k in range(8):
                sl = pl.ds(k * 16, 16)
                ev[i, sl] = jnp.maximum(xv[i, sl] + ev[i, sl], 0.0)
            return c2

        lax.fori_loop(0, B, edge, 0, unroll=2)
        pltpu.sync_copy(ev, accum.at[dstv], add=True)
        return carry

    lax.fori_loop(0, NB, block, 0)
    plsc.subcore_barrier()
    pltpu.sync_copy(accum.at[pl.ds(s * RPT, RPT)],
                    out_hbm.at[c, pl.ds(s * RPT, RPT)])


def _sc_aggregate(x, e, src, dst, zeros):
    mesh = plsc.VectorSubcoreMesh(core_axis_name="c", subcore_axis_name="s")
    f = pl.kernel(
        _sc_agg_body,
        out_type=jax.ShapeDtypeStruct((NC, N, D), jnp.float32),
        mesh=mesh,
        scratch_types=[
            pltpu.VMEM((B,), jnp.int32),
            pltpu.VMEM((B,), jnp.int32),
            pltpu.VMEM((B, D), jnp.float32),
            pltpu.VMEM((B, D), jnp.float32),
            pltpu.VMEM_SHARED((N, D), jnp.float32),
            pltpu.SemaphoreType.DMA,
        ],
    )
    return f(x, e, src, dst, zeros)


def _node_body(x_ref, a0_ref, a1_ref, w_ref, b_ref, out_ref):
    t = x_ref[...] + a0_ref[...] + a1_ref[...]
    out_ref[...] = jnp.maximum(
        jnp.dot(t, w_ref[...], preferred_element_type=jnp.float32) + b_ref[...],
        0.0,
    )


def _node_final_body(x_ref, a0_ref, a1_ref, w_ref, b_ref, wl_ref, bl_ref, out_ref):
    t = x_ref[...] + a0_ref[...] + a1_ref[...]
    h = jnp.maximum(
        jnp.dot(t, w_ref[...], preferred_element_type=jnp.float32) + b_ref[...],
        0.0,
    )
    out_ref[...] = (
        jnp.dot(h, wl_ref[...], preferred_element_type=jnp.float32) + bl_ref[...]
    )


_BN = 2000


def _node_mlp(x, a0, a1, W, b):
    return pl.pallas_call(
        _node_body,
        grid=(N // _BN,),
        in_specs=[
            pl.BlockSpec((_BN, D), lambda i: (i, 0)),
            pl.BlockSpec((_BN, D), lambda i: (i, 0)),
            pl.BlockSpec((_BN, D), lambda i: (i, 0)),
            pl.BlockSpec((D, D), lambda i: (0, 0)),
            pl.BlockSpec((1, D), lambda i: (0, 0)),
        ],
        out_specs=pl.BlockSpec((_BN, D), lambda i: (i, 0)),
        out_shape=jax.ShapeDtypeStruct((N, D), jnp.float32),
    )(x, a0, a1, W, b.reshape(1, D))


def _node_final(x, a0, a1, W, b, Wl, bl):
    return pl.pallas_call(
        _node_final_body,
        grid=(N // _BN,),
        in_specs=[
            pl.BlockSpec((_BN, D), lambda i: (i, 0)),
            pl.BlockSpec((_BN, D), lambda i: (i, 0)),
            pl.BlockSpec((_BN, D), lambda i: (i, 0)),
            pl.BlockSpec((D, D), lambda i: (0, 0)),
            pl.BlockSpec((1, D), lambda i: (0, 0)),
            pl.BlockSpec((D, D), lambda i: (0, 0)),
            pl.BlockSpec((1, D), lambda i: (0, 0)),
        ],
        out_specs=pl.BlockSpec((_BN, D), lambda i: (i, 0)),
        out_shape=jax.ShapeDtypeStruct((N, D), jnp.float32),
    )(x, a0, a1, W, b.reshape(1, D), Wl, bl.reshape(1, D))


def kernel(x, edge_index, edge_attr, We1, be1, W1, b1, We2, be2, W2, b2,
           We3, be3, W3, b3, Wl, bl):
    src = edge_index[0]
    dst = edge_index[1]
    zeros = jnp.zeros((RPT, D), jnp.float32)

    e = _edge_proj(edge_attr, We1, be1)
    a = _sc_aggregate(x, e, src, dst, zeros)
    h = _node_mlp(x, a[0], a[1], W1, b1)

    e = _edge_proj(edge_attr, We2, be2)
    a = _sc_aggregate(h, e, src, dst, zeros)
    h = _node_mlp(h, a[0], a[1], W2, b2)

    e = _edge_proj(edge_attr, We3, be3)
    a = _sc_aggregate(h, e, src, dst, zeros)
    return _node_final(h, a[0], a[1], W3, b3, Wl, bl)


# speedup vs baseline: 1.6801x; 1.6801x over previous
import jax
import jax.numpy as jnp
from jax import lax
from jax.experimental import pallas as pl
from jax.experimental.pallas import tpu as pltpu
from jax.experimental.pallas import tpu_sc as plsc

N = 10000
E = 320000
D = 128
ED = 16

NC = 2
NS = 16
NW = NC * NS
EPW = E // NW
B = 80
NB = EPW // B
NP = 10240
RPT = NP // NS


def _edge_proj_body(ea_ref, we_ref, be_ref, out_ref):
    out_ref[...] = (
        jnp.dot(ea_ref[...], we_ref[...], preferred_element_type=jnp.float32)
        + be_ref[...]
    )


def _edge_proj(edge_attr, We, be):
    BE = 4000
    return pl.pallas_call(
        _edge_proj_body,
        grid=(E // BE,),
        in_specs=[
            pl.BlockSpec((BE, ED), lambda i: (i, 0)),
            pl.BlockSpec((ED, D), lambda i: (0, 0)),
            pl.BlockSpec((1, D), lambda i: (0, 0)),
        ],
        out_specs=pl.BlockSpec((BE, D), lambda i: (i, 0)),
        out_shape=jax.ShapeDtypeStruct((E, D), jnp.float32),
    )(edge_attr, We, be.reshape(1, D))


def _sc_agg_body(x_hbm, e_hbm, src_hbm, dst_hbm, zero_hbm, out_hbm,
                 srcv, dstv, ev, xv, accum, sem):
    c = lax.axis_index("c")
    s = lax.axis_index("s")
    wid = c * NS + s

    pltpu.sync_copy(zero_hbm, accum.at[pl.ds(s * RPT, RPT)])
    plsc.subcore_barrier()

    ebase = pl.multiple_of(wid * EPW, 16)

    def block(j, carry):
        base = pl.multiple_of(ebase + j * B, 16)
        pltpu.sync_copy(src_hbm.at[pl.ds(base, B)], srcv)
        pltpu.sync_copy(dst_hbm.at[pl.ds(base, B)], dstv)
        pltpu.sync_copy(e_hbm.at[pl.ds(base, B)], ev)
        pltpu.async_copy(x_hbm.at[srcv], xv, sem).wait()

        def edge(i, c2):
            for k in range(8):
                sl = pl.ds(k * 16, 16)
                ev[i, sl] = jnp.maximum(xv[i, sl] + ev[i, sl], 0.0)
            return c2

        lax.fori_loop(0, B, edge, 0, unroll=2)
        pltpu.sync_copy(ev, accum.at[dstv], add=True)
        return carry

    lax.fori_loop(0, NB, block, 0)
    plsc.subcore_barrier()
    pltpu.sync_copy(accum.at[pl.ds(s * RPT, RPT)],
                    out_hbm.at[c, pl.ds(s * RPT, RPT)])


def _sc_aggregate(x, e, src, dst, zeros):
    mesh = plsc.VectorSubcoreMesh(core_axis_name="c", subcore_axis_name="s")
    f = pl.kernel(
        _sc_agg_body,
        out_type=jax.ShapeDtypeStruct((NC, NP, D), jnp.float32),
        mesh=mesh,
        scratch_types=[
            pltpu.VMEM((B,), jnp.int32),
            pltpu.VMEM((B,), jnp.int32),
            pltpu.VMEM((B, D), jnp.float32),
            pltpu.VMEM((B, D), jnp.float32),
            pltpu.VMEM_SHARED((NP, D), jnp.float32),
            pltpu.SemaphoreType.DMA,
        ],
    )
    return f(x, e, src, dst, zeros)


def _node_body(x_ref, a0_ref, a1_ref, w_ref, b_ref, out_ref):
    t = x_ref[...] + a0_ref[...] + a1_ref[...]
    out_ref[...] = jnp.maximum(
        jnp.dot(t, w_ref[...], preferred_element_type=jnp.float32) + b_ref[...],
        0.0,
    )


def _node_final_body(x_ref, a0_ref, a1_ref, w_ref, b_ref, wl_ref, bl_ref, out_ref):
    t = x_ref[...] + a0_ref[...] + a1_ref[...]
    h = jnp.maximum(
        jnp.dot(t, w_ref[...], preferred_element_type=jnp.float32) + b_ref[...],
        0.0,
    )
    out_ref[...] = (
        jnp.dot(h, wl_ref[...], preferred_element_type=jnp.float32) + bl_ref[...]
    )


_BN = 2000


def _node_mlp(x, a0, a1, W, b):
    return pl.pallas_call(
        _node_body,
        grid=(N // _BN,),
        in_specs=[
            pl.BlockSpec((_BN, D), lambda i: (i, 0)),
            pl.BlockSpec((_BN, D), lambda i: (i, 0)),
            pl.BlockSpec((_BN, D), lambda i: (i, 0)),
            pl.BlockSpec((D, D), lambda i: (0, 0)),
            pl.BlockSpec((1, D), lambda i: (0, 0)),
        ],
        out_specs=pl.BlockSpec((_BN, D), lambda i: (i, 0)),
        out_shape=jax.ShapeDtypeStruct((N, D), jnp.float32),
    )(x, a0, a1, W, b.reshape(1, D))


def _node_final(x, a0, a1, W, b, Wl, bl):
    return pl.pallas_call(
        _node_final_body,
        grid=(N // _BN,),
        in_specs=[
            pl.BlockSpec((_BN, D), lambda i: (i, 0)),
            pl.BlockSpec((_BN, D), lambda i: (i, 0)),
            pl.BlockSpec((_BN, D), lambda i: (i, 0)),
            pl.BlockSpec((D, D), lambda i: (0, 0)),
            pl.BlockSpec((1, D), lambda i: (0, 0)),
            pl.BlockSpec((D, D), lambda i: (0, 0)),
            pl.BlockSpec((1, D), lambda i: (0, 0)),
        ],
        out_specs=pl.BlockSpec((_BN, D), lambda i: (i, 0)),
        out_shape=jax.ShapeDtypeStruct((N, D), jnp.float32),
    )(x, a0, a1, W, b.reshape(1, D), Wl, bl.reshape(1, D))


def kernel(x, edge_index, edge_attr, We1, be1, W1, b1, We2, be2, W2, b2,
           We3, be3, W3, b3, Wl, bl):
    src = edge_index[0]
    dst = edge_index[1]
    zeros = jnp.zeros((RPT, D), jnp.float32)

    e = _edge_proj(edge_attr, We1, be1)
    a = _sc_aggregate(x, e, src, dst, zeros)
    h = _node_mlp(x, a[0], a[1], W1, b1)

    e = _edge_proj(edge_attr, We2, be2)
    a = _sc_aggregate(h, e, src, dst, zeros)
    h = _node_mlp(h, a[0], a[1], W2, b2)

    e = _edge_proj(edge_attr, We3, be3)
    a = _sc_aggregate(h, e, src, dst, zeros)
    return _node_final(h, a[0], a[1], W3, b3, Wl, bl)
